# baseline (speedup 1.0000x reference)
import functools

import jax
import jax.numpy as jnp
import numpy as np
from jax import lax
from jax.experimental import pallas as pl
from jax.experimental.pallas import tpu as pltpu
from jax.experimental.pallas import tpu_sc as plsc

N_NODES = 10000
NPAD = 10240
TIME_DIM = 100
B = 4096
K = 20
BK = B * K
QD = 356
DH = QD // 2
P = 192
QP = 2 * P
KVW = 2 * QP
QSW = QP + 256
BLK_A = 512
BB = 128
NW = 32
ROWS_W = BK // NW
SRC_W = B // NW
F32 = jnp.float32
BF16 = jnp.bfloat16


def _padhead(m):
    z = jnp.zeros((m.shape[0], QP), F32)
    z = z.at[:, :DH].set(m[:, :DH])
    z = z.at[:, P:P + DH].set(m[:, DH:])
    return z


def _tables_body(nf, mem, pos, p1t, p1b, p2t, p2b, w1a, w1b, b1, w11t, b11,
                 w2a, w2b, b2, w22t, b22, wq, cq, wk, ck, wv, cv,
                 kv_o, qs_o):
    dot = functools.partial(jnp.dot, preferred_element_type=F32)
    feats = nf[...] + mem[...]
    h = jnp.maximum(dot(pos[...], p1t[...]) + p1b[...], 0.0)
    pe = dot(h, p2t[...]) + p2b[...]
    h1 = jnp.maximum(dot(feats, w1a[...]) + dot(pe, w1b[...]) + b1[...], 0.0)
    h1 = dot(h1, w11t[...]) + b11[...]
    h2 = jnp.maximum(dot(feats, w2a[...]) + dot(pe, w2b[...]) + b2[...], 0.0)
    h2 = dot(h2, w22t[...]) + b22[...]
    qs_o[:, :QP] = dot(h1, wq[...]) + cq[...]
    qs_o[:, QP:] = h1
    kv_o[:, :QP] = (dot(h2, wk[...]) + ck[...]).astype(BF16)
    kv_o[:, QP:] = (dot(h2, wv[...]) + cv[...]).astype(BF16)


def _attn_body(qs, kv, er, delta, nbr, tw, tb, wt, we, wout, outb,
               m1a, m1b, mb1, m2t, mb2, out_o):
    dot = functools.partial(jnp.dot, preferred_element_type=F32)
    t_emb = jnp.cos(delta[...] * tw[...] + tb[...]).astype(BF16)
    kvf = (kv[...].astype(F32) + dot(t_emb, wt[...])
           + dot(er[:, :16].astype(BF16), we[...]))
    kv3 = kvf.reshape(BB, K, KVW)
    q = qs[:, 0:QP]
    s = qs[:, QP:QSW]
    qk = q[:, None, :] * kv3[:, :, 0:QP]
    scale = np.float32(1.0 / np.sqrt(DH))
    s0 = jnp.sum(qk[:, :, 0:P], axis=-1) * scale
    s1 = jnp.sum(qk[:, :, P:QP], axis=-1) * scale
    nb = nbr[...]
    mask = nb == 0
    invalid = jnp.all(mask, axis=1, keepdims=True)
    col = lax.broadcasted_iota(jnp.int32, (BB, K), 1)
    mask = jnp.logical_and(mask,
                           jnp.logical_not(jnp.logical_and(invalid, col == 0)))
    s0 = jnp.where(mask, -1e30, s0)
    s1 = jnp.where(mask, -1e30, s1)

    def _softmax(x):
        m = jnp.max(x, axis=1, keepdims=True)
        e = jnp.exp(x - m)
        return e / jnp.sum(e, axis=1, keepdims=True)

    a0 = _softmax(s0)
    a1 = _softmax(s1)
    ctx0 = jnp.sum(a0[:, :, None] * kv3[:, :, QP:QP + P], axis=1)
    ctx1 = jnp.sum(a1[:, :, None] * kv3[:, :, QP + P:KVW], axis=1)
    ctx = jnp.concatenate([ctx0, ctx1], axis=1)
    ao = dot(ctx, wout[...]) + outb[...]
    ao = jnp.where(invalid, 0.0, ao)
    hm = jnp.maximum(dot(ao, m1a[...]) + dot(s, m1b[...]) + mb1[...], 0.0)
    out_o[...] = dot(hm, m2t[...]) + mb2[...]


def _sc_gather(kv_tab, e_tab, qs_tab, nb_idx, ei_idx, src_idx):
    mesh = plsc.VectorSubcoreMesh(core_axis_name="c", subcore_axis_name="s")

    @functools.partial(
        pl.kernel, mesh=mesh,
        out_type=(jax.ShapeDtypeStruct((BK, QP), F32),
                  jax.ShapeDtypeStruct((BK, 128), F32),
                  jax.ShapeDtypeStruct((B, QSW), F32)),
        scratch_types=[pltpu.VMEM((ROWS_W,), jnp.int32),
                       pltpu.VMEM((ROWS_W,), jnp.int32),
                       pltpu.VMEM((SRC_W,), jnp.int32),
                       pltpu.VMEM((128, QP), F32),
                       pltpu.VMEM((128, 128), F32),
                       pltpu.VMEM((64, QSW), F32),
                       pltpu.SemaphoreType.DMA],
    )
    def k(kv_t, e_t, qs_t, nbi, eii, srci, kv_o, e_o, qs_o,
          nbv, eiv, srcv, kvbuf, ebuf, qsbuf, sem):
        wid = lax.axis_index("s") * 2 + lax.axis_index("c")
        bn = wid * ROWS_W
        bb = wid * SRC_W
        pltpu.sync_copy(nbi.at[pl.ds(bn, ROWS_W)], nbv)
        pltpu.sync_copy(eii.at[pl.ds(bn, ROWS_W)], eiv)
        pltpu.sync_copy(srci.at[pl.ds(bb, SRC_W)], srcv)

        def kv_chunk(c, carry):
            pltpu.async_copy(kv_t.at[nbv.at[pl.ds(c * 128, 128)]], kvbuf, sem).wait()
            pltpu.sync_copy(kvbuf, kv_o.at[pl.ds(bn + c * 128, 128)])
            return carry

        lax.fori_loop(0, ROWS_W // 128, kv_chunk, 0)

        def e_chunk(c, carry):
            pltpu.async_copy(e_t.at[eiv.at[pl.ds(c * 128, 128)]], ebuf, sem).wait()
            pltpu.sync_copy(ebuf, e_o.at[pl.ds(bn + c * 128, 128)])
            return carry

        lax.fori_loop(0, ROWS_W // 128, e_chunk, 0)

        def qs_chunk(c, carry):
            pltpu.async_copy(qs_t.at[srcv.at[pl.ds(c * 64, 64)]], qsbuf, sem).wait()
            pltpu.sync_copy(qsbuf, qs_o.at[pl.ds(bb + c * 64, 64)])
            return carry

        lax.fori_loop(0, SRC_W // 64, qs_chunk, 0)

    return k(kv_tab, e_tab, qs_tab, nb_idx, ei_idx, src_idx)


def kernel(params, node_features, edge_features, memory, position_memory,
           timestamps, edge_times, source_nodes, neighbors, edge_idxs):
    p = params
    padn = ((0, NPAD - N_NODES), (0, 0))
    nf = jnp.pad(node_features, padn)
    mem = jnp.pad(memory, padn)
    pos = jnp.pad(position_memory, padn)

    row = lambda v: v[None, :].astype(F32)
    p1t = p['pos_w1'].T
    p2t = p['pos_w2'].T
    w1a = p['lin1_w'][:, :256].T
    w1b = p['lin1_w'][:, 256:].T
    w11t = p['lin11_w'].T
    w2a = p['lin2_w'][:, :256].T
    w2b = p['lin2_w'][:, 256:].T
    w22t = p['lin22_w'].T
    tconst = jnp.cos(p['time_b'])
    wq = _padhead(p['q_w'][:, :256].T)
    cq = _padhead((tconst @ p['q_w'][:, 256:].T + p['q_b'])[None, :])
    wk = _padhead(p['k_w'][:, :256].T)
    ck = _padhead(p['k_b'][None, :])
    wv = _padhead(p['v_w'][:, :256].T)
    cv = _padhead(p['v_b'][None, :])

    grid_a = NPAD // BLK_A
    full = lambda shape: pl.BlockSpec(shape, lambda i: (0, 0))
    blk = lambda w: pl.BlockSpec((BLK_A, w), lambda i: (i, 0))
    kv_tab, qs_tab = pl.pallas_call(
        _tables_body,
        grid=(grid_a,),
        in_specs=[blk(256), blk(256), blk(8),
                  full((8, 16)), full((1, 16)), full((16, 12)), full((1, 12)),
                  full((256, 256)), full((12, 256)), full((1, 256)),
                  full((256, 256)), full((1, 256)),
                  full((256, 256)), full((12, 256)), full((1, 256)),
                  full((256, 256)), full((1, 256)),
                  full((256, QP)), full((1, QP)),
                  full((256, QP)), full((1, QP)),
                  full((256, QP)), full((1, QP))],
        out_specs=[blk(KVW), blk(QSW)],
        out_shape=[jax.ShapeDtypeStruct((NPAD, KVW), BF16),
                   jax.ShapeDtypeStruct((NPAD, QSW), F32)],
    )(nf, mem, pos, p1t, row(p['pos_b1']), p2t, row(p['pos_b2']),
      w1a, w1b, row(p['lin1_b']), w11t, row(p['lin11_b']),
      w2a, w2b, row(p['lin2_b']), w22t, row(p['lin22_b']),
      wq, cq, wk, ck, wv, cv)

    kv_bits = lax.bitcast_convert_type(kv_tab.reshape(NPAD, QP, 2), F32)
    nb_flat = neighbors.reshape(-1).astype(jnp.int32)
    ei_flat = edge_idxs.reshape(-1).astype(jnp.int32)
    src = source_nodes.astype(jnp.int32)
    ef = jnp.pad(edge_features, ((0, 0), (0, 112)))
    kv_rows_bits, e_rows, qs_rows = _sc_gather(kv_bits, ef, qs_tab,
                                               nb_flat, ei_flat, src)
    kv_rows = lax.bitcast_convert_type(kv_rows_bits, BF16).reshape(BK, KVW)

    delta = (timestamps[:, None] - edge_times).reshape(-1, 1)
    wt = jnp.concatenate([_padhead(p['k_w'][:, 272:].T),
                          _padhead(p['v_w'][:, 272:].T)], axis=1).astype(BF16)
    we = jnp.concatenate([_padhead(p['k_w'][:, 256:272].T),
                          _padhead(p['v_w'][:, 256:272].T)], axis=1).astype(BF16)
    wout = jnp.zeros((QP, QP), F32)
    owt = p['out_w'].T
    wout = wout.at[0:DH, 0:QD].set(owt[0:DH, :])
    wout = wout.at[P:P + DH, 0:QD].set(owt[DH:QD, :])
    outb = jnp.zeros((1, QP), F32).at[0, :QD].set(p['out_b'])
    m1a = jnp.zeros((QP, 256), F32).at[0:QD, :].set(p['mrg_w1'][:, :QD].T)
    m1b = p['mrg_w1'][:, QD:].T
    grid_c = B // BB
    fullc = lambda shape: pl.BlockSpec(shape, lambda i: (0, 0))
    out = pl.pallas_call(
        _attn_body,
        grid=(grid_c,),
        in_specs=[pl.BlockSpec((BB, QSW), lambda i: (i, 0)),
                  pl.BlockSpec((BB * K, KVW), lambda i: (i, 0)),
                  pl.BlockSpec((BB * K, 128), lambda i: (i, 0)),
                  pl.BlockSpec((BB * K, 1), lambda i: (i, 0)),
                  pl.BlockSpec((BB, K), lambda i: (i, 0)),
                  fullc((1, TIME_DIM)), fullc((1, TIME_DIM)),
                  fullc((TIME_DIM, KVW)), fullc((16, KVW)),
                  fullc((QP, QP)), fullc((1, QP)),
                  fullc((QP, 256)), fullc((256, 256)), fullc((1, 256)),
                  fullc((256, 256)), fullc((1, 256))],
        out_specs=[pl.BlockSpec((BB, 256), lambda i: (i, 0))],
        out_shape=[jax.ShapeDtypeStruct((B, 256), F32)],
    )(qs_rows, kv_rows, e_rows, delta, neighbors.astype(jnp.int32),
      row(p['time_w'][:, 0]), row(p['time_b']), wt, we, wout, outb,
      m1a, m1b, row(p['mrg_b1']), p['mrg_w2'].T, row(p['mrg_b2']))[0]
    return out


# speedup vs baseline: 2.8115x; 2.8115x over previous
import functools

import jax
import jax.numpy as jnp
import numpy as np
from jax import lax
from jax.experimental import pallas as pl
from jax.experimental.pallas import tpu as pltpu
from jax.experimental.pallas import tpu_sc as plsc

N_NODES = 10000
NPAD = 10240
TIME_DIM = 100
B = 4096
K = 20
BK = B * K
QD = 356
DH = QD // 2
P = 192
QP = 2 * P
KVW = 2 * QP
QSW = QP + 256
BLK_A = 512
BB = 128
NW = 32
ROWS_W = BK // NW
SRC_W = B // NW
F32 = jnp.float32
BF16 = jnp.bfloat16


def _padhead(m):
    z = jnp.zeros((m.shape[0], QP), F32)
    z = z.at[:, :DH].set(m[:, :DH])
    z = z.at[:, P:P + DH].set(m[:, DH:])
    return z


def _tables_body(nf, mem, pos, p1t, p1b, p2t, p2b, w1a, w1b, b1, w11t, b11,
                 w2a, w2b, b2, w22t, b22, wq, cq, wk, ck, wv, cv,
                 kv_o, qs_o):
    dot = functools.partial(jnp.dot, preferred_element_type=F32)
    feats = nf[...] + mem[...]
    h = jnp.maximum(dot(pos[...], p1t[...]) + p1b[...], 0.0)
    pe = dot(h, p2t[...]) + p2b[...]
    h1 = jnp.maximum(dot(feats, w1a[...]) + dot(pe, w1b[...]) + b1[...], 0.0)
    h1 = dot(h1, w11t[...]) + b11[...]
    h2 = jnp.maximum(dot(feats, w2a[...]) + dot(pe, w2b[...]) + b2[...], 0.0)
    h2 = dot(h2, w22t[...]) + b22[...]
    qs_o[:, :QP] = dot(h1, wq[...]) + cq[...]
    qs_o[:, QP:] = h1
    kb = (dot(h2, wk[...]) + ck[...]).astype(BF16).astype(F32)
    vb = (dot(h2, wv[...]) + cv[...]).astype(BF16).astype(F32)
    kbits = lax.bitcast_convert_type(kb, jnp.uint32)
    vbits = lax.bitcast_convert_type(vb, jnp.uint32)
    word = (kbits >> jnp.uint32(16)) | (vbits & jnp.uint32(0xFFFF0000))
    kv_o[...] = lax.bitcast_convert_type(word, F32)


def _attn_body(qs, kv, er, delta, nbr, tw, tb, wtk, wtv, wek, wev, wout, outb,
               m1a, m1b, mb1, m2t, mb2, out_o):
    dot = functools.partial(jnp.dot, preferred_element_type=F32)
    t_emb = jnp.cos(delta[...] * tw[...] + tb[...]).astype(BF16)
    bits = lax.bitcast_convert_type(kv[...], jnp.uint32)
    er16 = er[:, :16].astype(BF16)
    kf = (lax.bitcast_convert_type(bits << jnp.uint32(16), F32)
          + dot(t_emb, wtk[...]) + dot(er16, wek[...]))
    vf = (lax.bitcast_convert_type(bits & jnp.uint32(0xFFFF0000), F32)
          + dot(t_emb, wtv[...]) + dot(er16, wev[...]))
    k3 = kf.reshape(BB, K, QP)
    v3 = vf.reshape(BB, K, QP)
    q = qs[:, 0:QP]
    s = qs[:, QP:QSW]
    qk = q[:, None, :] * k3
    scale = np.float32(1.0 / np.sqrt(DH))
    s0 = jnp.sum(qk[:, :, 0:P], axis=-1) * scale
    s1 = jnp.sum(qk[:, :, P:QP], axis=-1) * scale
    nb = nbr[...]
    mask = nb == 0
    invalid = jnp.all(mask, axis=1, keepdims=True)
    col = lax.broadcasted_iota(jnp.int32, (BB, K), 1)
    mask = jnp.logical_and(mask,
                           jnp.logical_not(jnp.logical_and(invalid, col == 0)))
    s0 = jnp.where(mask, -1e30, s0)
    s1 = jnp.where(mask, -1e30, s1)

    def _softmax(x):
        m = jnp.max(x, axis=1, keepdims=True)
        e = jnp.exp(x - m)
        return e / jnp.sum(e, axis=1, keepdims=True)

    a0 = _softmax(s0)
    a1 = _softmax(s1)
    ctx0 = jnp.sum(a0[:, :, None] * v3[:, :, 0:P], axis=1)
    ctx1 = jnp.sum(a1[:, :, None] * v3[:, :, P:QP], axis=1)
    ctx = jnp.concatenate([ctx0, ctx1], axis=1)
    ao = dot(ctx, wout[...]) + outb[...]
    ao = jnp.where(invalid, 0.0, ao)
    hm = jnp.maximum(dot(ao, m1a[...]) + dot(s, m1b[...]) + mb1[...], 0.0)
    out_o[...] = dot(hm, m2t[...]) + mb2[...]


def _sc_gather(kv_tab, e_tab, qs_tab, nb_idx, ei_idx, src_idx):
    mesh = plsc.VectorSubcoreMesh(core_axis_name="c", subcore_axis_name="s")

    @functools.partial(
        pl.kernel, mesh=mesh,
        out_type=(jax.ShapeDtypeStruct((BK, QP), F32),
                  jax.ShapeDtypeStruct((BK, 128), F32),
                  jax.ShapeDtypeStruct((B, QSW), F32)),
        scratch_types=[pltpu.VMEM((ROWS_W,), jnp.int32),
                       pltpu.VMEM((ROWS_W,), jnp.int32),
                       pltpu.VMEM((SRC_W,), jnp.int32),
                       pltpu.VMEM((128, QP), F32),
                       pltpu.VMEM((128, 128), F32),
                       pltpu.VMEM((64, QSW), F32),
                       pltpu.SemaphoreType.DMA],
    )
    def k(kv_t, e_t, qs_t, nbi, eii, srci, kv_o, e_o, qs_o,
          nbv, eiv, srcv, kvbuf, ebuf, qsbuf, sem):
        wid = lax.axis_index("s") * 2 + lax.axis_index("c")
        bn = wid * ROWS_W
        bb = wid * SRC_W
        pltpu.sync_copy(nbi.at[pl.ds(bn, ROWS_W)], nbv)
        pltpu.sync_copy(eii.at[pl.ds(bn, ROWS_W)], eiv)
        pltpu.sync_copy(srci.at[pl.ds(bb, SRC_W)], srcv)

        def kv_chunk(c, carry):
            pltpu.async_copy(kv_t.at[nbv.at[pl.ds(c * 128, 128)]], kvbuf, sem).wait()
            pltpu.sync_copy(kvbuf, kv_o.at[pl.ds(bn + c * 128, 128)])
            return carry

        lax.fori_loop(0, ROWS_W // 128, kv_chunk, 0)

        def e_chunk(c, carry):
            pltpu.async_copy(e_t.at[eiv.at[pl.ds(c * 128, 128)]], ebuf, sem).wait()
            pltpu.sync_copy(ebuf, e_o.at[pl.ds(bn + c * 128, 128)])
            return carry

        lax.fori_loop(0, ROWS_W // 128, e_chunk, 0)

        def qs_chunk(c, carry):
            pltpu.async_copy(qs_t.at[srcv.at[pl.ds(c * 64, 64)]], qsbuf, sem).wait()
            pltpu.sync_copy(qsbuf, qs_o.at[pl.ds(bb + c * 64, 64)])
            return carry

        lax.fori_loop(0, SRC_W // 64, qs_chunk, 0)

    return k(kv_tab, e_tab, qs_tab, nb_idx, ei_idx, src_idx)


def kernel(params, node_features, edge_features, memory, position_memory,
           timestamps, edge_times, source_nodes, neighbors, edge_idxs):
    p = params
    padn = ((0, NPAD - N_NODES), (0, 0))
    nf = jnp.pad(node_features, padn)
    mem = jnp.pad(memory, padn)
    pos = jnp.pad(position_memory, padn)

    row = lambda v: v[None, :].astype(F32)
    p1t = p['pos_w1'].T
    p2t = p['pos_w2'].T
    w1a = p['lin1_w'][:, :256].T
    w1b = p['lin1_w'][:, 256:].T
    w11t = p['lin11_w'].T
    w2a = p['lin2_w'][:, :256].T
    w2b = p['lin2_w'][:, 256:].T
    w22t = p['lin22_w'].T
    tconst = jnp.cos(p['time_b'])
    wq = _padhead(p['q_w'][:, :256].T)
    cq = _padhead((tconst @ p['q_w'][:, 256:].T + p['q_b'])[None, :])
    wk = _padhead(p['k_w'][:, :256].T)
    ck = _padhead(p['k_b'][None, :])
    wv = _padhead(p['v_w'][:, :256].T)
    cv = _padhead(p['v_b'][None, :])

    grid_a = NPAD // BLK_A
    full = lambda shape: pl.BlockSpec(shape, lambda i: (0, 0))
    blk = lambda w: pl.BlockSpec((BLK_A, w), lambda i: (i, 0))
    kv_tab, qs_tab = pl.pallas_call(
        _tables_body,
        grid=(grid_a,),
        in_specs=[blk(256), blk(256), blk(8),
                  full((8, 16)), full((1, 16)), full((16, 12)), full((1, 12)),
                  full((256, 256)), full((12, 256)), full((1, 256)),
                  full((256, 256)), full((1, 256)),
                  full((256, 256)), full((12, 256)), full((1, 256)),
                  full((256, 256)), full((1, 256)),
                  full((256, QP)), full((1, QP)),
                  full((256, QP)), full((1, QP)),
                  full((256, QP)), full((1, QP))],
        out_specs=[blk(QP), blk(QSW)],
        out_shape=[jax.ShapeDtypeStruct((NPAD, QP), F32),
                   jax.ShapeDtypeStruct((NPAD, QSW), F32)],
    )(nf, mem, pos, p1t, row(p['pos_b1']), p2t, row(p['pos_b2']),
      w1a, w1b, row(p['lin1_b']), w11t, row(p['lin11_b']),
      w2a, w2b, row(p['lin2_b']), w22t, row(p['lin22_b']),
      wq, cq, wk, ck, wv, cv)

    nb_flat = neighbors.reshape(-1).astype(jnp.int32)
    ei_flat = edge_idxs.reshape(-1).astype(jnp.int32)
    src = source_nodes.astype(jnp.int32)
    ef = jnp.pad(edge_features, ((0, 0), (0, 112)))
    kv_rows, e_rows, qs_rows = _sc_gather(kv_tab, ef, qs_tab,
                                          nb_flat, ei_flat, src)

    delta = (timestamps[:, None] - edge_times).reshape(-1, 1)
    wtk = _padhead(p['k_w'][:, 272:].T).astype(BF16)
    wtv = _padhead(p['v_w'][:, 272:].T).astype(BF16)
    wek = _padhead(p['k_w'][:, 256:272].T).astype(BF16)
    wev = _padhead(p['v_w'][:, 256:272].T).astype(BF16)
    wout = jnp.zeros((QP, QP), F32)
    owt = p['out_w'].T
    wout = wout.at[0:DH, 0:QD].set(owt[0:DH, :])
    wout = wout.at[P:P + DH, 0:QD].set(owt[DH:QD, :])
    outb = jnp.zeros((1, QP), F32).at[0, :QD].set(p['out_b'])
    m1a = jnp.zeros((QP, 256), F32).at[0:QD, :].set(p['mrg_w1'][:, :QD].T)
    m1b = p['mrg_w1'][:, QD:].T
    grid_c = B // BB
    fullc = lambda shape: pl.BlockSpec(shape, lambda i: (0, 0))
    out = pl.pallas_call(
        _attn_body,
        grid=(grid_c,),
        in_specs=[pl.BlockSpec((BB, QSW), lambda i: (i, 0)),
                  pl.BlockSpec((BB * K, QP), lambda i: (i, 0)),
                  pl.BlockSpec((BB * K, 128), lambda i: (i, 0)),
                  pl.BlockSpec((BB * K, 1), lambda i: (i, 0)),
                  pl.BlockSpec((BB, K), lambda i: (i, 0)),
                  fullc((1, TIME_DIM)), fullc((1, TIME_DIM)),
                  fullc((TIME_DIM, QP)), fullc((TIME_DIM, QP)),
                  fullc((16, QP)), fullc((16, QP)),
                  fullc((QP, QP)), fullc((1, QP)),
                  fullc((QP, 256)), fullc((256, 256)), fullc((1, 256)),
                  fullc((256, 256)), fullc((1, 256))],
        out_specs=[pl.BlockSpec((BB, 256), lambda i: (i, 0))],
        out_shape=[jax.ShapeDtypeStruct((B, 256), F32)],
    )(qs_rows, kv_rows, e_rows, delta, neighbors.astype(jnp.int32),
      row(p['time_w'][:, 0]), row(p['time_b']), wtk, wtv, wek, wev, wout, outb,
      m1a, m1b, row(p['mrg_b1']), p['mrg_w2'].T, row(p['mrg_b2']))[0]
    return out
